# baseline (speedup 1.0000x reference)
import functools

import jax
import jax.numpy as jnp
from jax import lax
from jax.experimental import pallas as pl
from jax.experimental.pallas import tpu as pltpu
from jax.experimental.pallas import tpu_sc as plsc

N = 10000
E = 320000
D = 128
NC = 2
NS = 16
NTILES = NC * NS
NP = ((N + 255) // 256) * 256
RPT = NP // NS
C = 128
NCHUNK = E // C
CH = NCHUNK // NTILES
XTRA = NCHUNK - CH * NTILES
BLK = 1024
GRID = (N + BLK - 1) // BLK

_mesh = plsc.VectorSubcoreMesh(core_axis_name="c", subcore_axis_name="s")


@functools.partial(
    pl.kernel,
    mesh=_mesh,
    out_type=jax.ShapeDtypeStruct((NC, NP), jnp.float32),
    scratch_types=[
        pltpu.VMEM((2, C), jnp.int32),
        pltpu.VMEM((2, C), jnp.int32),
        pltpu.VMEM((C,), jnp.float32),
        pltpu.VMEM((RPT,), jnp.float32),
        pltpu.VMEM_SHARED((NP,), jnp.float32),
        pltpu.SemaphoreType.DMA,
        pltpu.SemaphoreType.DMA,
    ],
)
def _deg_sc(eH_hbm, out_hbm, idx_a, idx_b, ones_v, zbuf, acc, sem_a, sem_b):
    cid = lax.axis_index("c")
    sid = lax.axis_index("s")
    wid = sid * NC + cid
    base = wid * CH + jnp.minimum(wid, XTRA)
    extra = wid < XTRA

    def fill_ones(i, carry):
        ones_v[pl.ds(i * 16, 16)] = jnp.ones((16,), jnp.float32)
        return carry

    lax.fori_loop(0, C // 16, fill_ones, 0)

    def fill_zeros(i, carry):
        zbuf[pl.ds(i * 16, 16)] = jnp.zeros((16,), jnp.float32)
        return carry

    lax.fori_loop(0, RPT // 16, fill_zeros, 0)
    pltpu.sync_copy(zbuf, acc.at[pl.ds(sid * RPT, RPT)])
    plsc.subcore_barrier()

    def istart(c, buf, sem):
        pltpu.async_copy(eH_hbm.at[:, pl.ds((base + c) * C, C)], buf, sem)

    def iwait(c, buf, sem):
        pltpu.make_async_copy(eH_hbm.at[:, pl.ds((base + c) * C, C)],
                              buf, sem).wait()

    def scat(buf):
        pltpu.sync_copy(ones_v, acc.at[buf.at[1]], add=True)

    istart(0, idx_a, sem_a)

    def body(i, carry):
        c0 = 2 * i
        istart(c0 + 1, idx_b, sem_b)
        iwait(c0, idx_a, sem_a)
        scat(idx_a)
        istart(c0 + 2, idx_a, sem_a)
        iwait(c0 + 1, idx_b, sem_b)
        scat(idx_b)
        return carry

    lax.fori_loop(0, CH // 2 - 1, body, 0)
    istart(CH - 1, idx_b, sem_b)
    iwait(CH - 2, idx_a, sem_a)
    scat(idx_a)
    iwait(CH - 1, idx_b, sem_b)
    scat(idx_b)

    @pl.when(extra)
    def _():
        istart(CH, idx_a, sem_a)
        iwait(CH, idx_a, sem_a)
        scat(idx_a)

    plsc.subcore_barrier()
    pltpu.sync_copy(acc.at[pl.ds(sid * RPT, RPT)],
                    out_hbm.at[cid, pl.ds(sid * RPT, RPT)])


def _mm_body(deg_ref, x_ref, w_ref, y_ref):
    d = deg_ref[0] + deg_ref[1] + 1.0
    dinv = lax.rsqrt(jnp.maximum(d, 1e-12))
    xw = jnp.dot(x_ref[...], w_ref[...], preferred_element_type=jnp.float32)
    y_ref[...] = xw * dinv[:, None]


def _mm_tc(degp, xp, W):
    return pl.pallas_call(
        _mm_body,
        grid=(GRID,),
        in_specs=[
            pl.BlockSpec((NC, BLK), lambda i: (0, i)),
            pl.BlockSpec((BLK, D), lambda i: (i, 0)),
            pl.BlockSpec((D, D), lambda i: (0, 0)),
        ],
        out_specs=pl.BlockSpec((BLK, D), lambda i: (i, 0)),
        out_shape=jax.ShapeDtypeStruct((N, D), jnp.float32),
    )(degp, xp, W)


@functools.partial(
    pl.kernel,
    mesh=_mesh,
    out_type=jax.ShapeDtypeStruct((NC, NP, D), jnp.float32),
    scratch_types=[
        pltpu.VMEM((2, C), jnp.int32),
        pltpu.VMEM((2, C), jnp.int32),
        pltpu.VMEM((C, D), jnp.float32),
        pltpu.VMEM((C, D), jnp.float32),
        pltpu.VMEM((16, D), jnp.float32),
        pltpu.VMEM_SHARED((NP, D), jnp.float32),
        pltpu.SemaphoreType.DMA,
        pltpu.SemaphoreType.DMA,
        pltpu.SemaphoreType.DMA,
        pltpu.SemaphoreType.DMA,
    ],
)
def _scat_sc(y_hbm, eH_hbm, out_hbm, idx_a, idx_b, rows_a, rows_b,
             zbuf, acc, gsem_a, gsem_b, isem_a, isem_b):
    cid = lax.axis_index("c")
    sid = lax.axis_index("s")
    wid = sid * NC + cid
    base = wid * CH + jnp.minimum(wid, XTRA)
    extra = wid < XTRA

    for i in range(16):
        for j in range(D // 16):
            zbuf[i, pl.ds(j * 16, 16)] = jnp.zeros((16,), jnp.float32)

    def zero_rows(i, carry):
        pltpu.sync_copy(zbuf, acc.at[pl.ds(sid * RPT + i * 16, 16)])
        return carry

    lax.fori_loop(0, RPT // 16, zero_rows, 0)
    plsc.subcore_barrier()

    def istart(c, buf, sem):
        pltpu.async_copy(eH_hbm.at[:, pl.ds((base + c) * C, C)], buf, sem)

    def iwait(c, buf, sem):
        pltpu.make_async_copy(eH_hbm.at[:, pl.ds((base + c) * C, C)],
                              buf, sem).wait()

    def gstart(ibuf, rbuf, sem):
        pltpu.async_copy(y_hbm.at[ibuf.at[0]], rbuf, sem)

    def gwait(ibuf, rbuf, sem):
        pltpu.make_async_copy(y_hbm.at[ibuf.at[0]], rbuf, sem).wait()

    def scat(ibuf, rbuf):
        pltpu.sync_copy(rbuf, acc.at[ibuf.at[1]], add=True)

    istart(0, idx_a, isem_a)
    iwait(0, idx_a, isem_a)
    gstart(idx_a, rows_a, gsem_a)
    istart(1, idx_b, isem_b)

    def body(i, carry):
        c0 = 2 * i
        iwait(c0 + 1, idx_b, isem_b)
        gstart(idx_b, rows_b, gsem_b)
        gwait(idx_a, rows_a, gsem_a)
        scat(idx_a, rows_a)
        istart(c0 + 2, idx_a, isem_a)
        iwait(c0 + 2, idx_a, isem_a)
        gstart(idx_a, rows_a, gsem_a)
        gwait(idx_b, rows_b, gsem_b)
        scat(idx_b, rows_b)
        istart(c0 + 3, idx_b, isem_b)
        return carry

    lax.fori_loop(0, CH // 2 - 1, body, 0)
    iwait(CH - 1, idx_b, isem_b)
    gstart(idx_b, rows_b, gsem_b)
    gwait(idx_a, rows_a, gsem_a)
    scat(idx_a, rows_a)
    gwait(idx_b, rows_b, gsem_b)
    scat(idx_b, rows_b)

    @pl.when(extra)
    def _():
        istart(CH, idx_a, isem_a)
        iwait(CH, idx_a, isem_a)
        gstart(idx_a, rows_a, gsem_a)
        gwait(idx_a, rows_a, gsem_a)
        scat(idx_a, rows_a)

    plsc.subcore_barrier()
    pltpu.sync_copy(acc.at[pl.ds(sid * RPT, RPT)],
                    out_hbm.at[cid, pl.ds(sid * RPT, RPT)])


def _fin_body(sp_ref, y_ref, deg_ref, b_ref, o_ref):
    d = deg_ref[0] + deg_ref[1] + 1.0
    dinv = lax.rsqrt(jnp.maximum(d, 1e-12))
    s = sp_ref[0] + sp_ref[1] + y_ref[...]
    o_ref[...] = dinv[:, None] * s + b_ref[0][None, :]


def _fin_tc(Sp, y, degp, b2):
    return pl.pallas_call(
        _fin_body,
        grid=(GRID,),
        in_specs=[
            pl.BlockSpec((NC, BLK, D), lambda i: (0, i, 0)),
            pl.BlockSpec((BLK, D), lambda i: (i, 0)),
            pl.BlockSpec((NC, BLK), lambda i: (0, i)),
            pl.BlockSpec((1, D), lambda i: (0, 0)),
        ],
        out_specs=pl.BlockSpec((BLK, D), lambda i: (i, 0)),
        out_shape=jax.ShapeDtypeStruct((N, D), jnp.float32),
    )(Sp, y, degp, b2)


def kernel(x, edge_index, W, b):
    degp = _deg_sc(edge_index)
    y = _mm_tc(degp, x, W)
    Sp = _scat_sc(y, edge_index)
    return _fin_tc(Sp, y, degp, b.reshape(1, D))


# speedup vs baseline: 1.1466x; 1.1466x over previous
import functools

import jax
import jax.numpy as jnp
from jax import lax
from jax.experimental import pallas as pl
from jax.experimental.pallas import tpu as pltpu
from jax.experimental.pallas import tpu_sc as plsc

N = 10000
E = 320000
D = 128
NC = 2
NS = 16
NTILES = NC * NS
NP = ((N + 255) // 256) * 256
RPT = NP // NS
C = 125
NB = 10
BC = 8
CH = NB * BC
EPT = CH * C
BLK = 1024
GRID = (N + BLK - 1) // BLK

_mesh = plsc.VectorSubcoreMesh(core_axis_name="c", subcore_axis_name="s")


@functools.partial(
    pl.kernel,
    mesh=_mesh,
    out_type=jax.ShapeDtypeStruct((NC, NP), jnp.float32),
    scratch_types=[
        pltpu.VMEM((NB, BC, C), jnp.int32),
        pltpu.VMEM((128,), jnp.float32),
        pltpu.VMEM((RPT,), jnp.float32),
        pltpu.VMEM_SHARED((NP,), jnp.float32),
    ],
)
def _deg_sc(eS_hbm, out_hbm, dst_v, ones_v, zbuf, acc):
    cid = lax.axis_index("c")
    sid = lax.axis_index("s")
    wid = sid * NC + cid

    def fill_ones(i, carry):
        ones_v[pl.ds(i * 16, 16)] = jnp.ones((16,), jnp.float32)
        return carry

    lax.fori_loop(0, 8, fill_ones, 0)

    def fill_zeros(i, carry):
        zbuf[pl.ds(i * 16, 16)] = jnp.zeros((16,), jnp.float32)
        return carry

    lax.fori_loop(0, RPT // 16, fill_zeros, 0)
    pltpu.sync_copy(zbuf, acc.at[pl.ds(sid * RPT, RPT)])
    pltpu.sync_copy(eS_hbm.at[1, wid], dst_v)
    plsc.subcore_barrier()

    ones = ones_v.at[pl.ds(0, C)]

    def body(j, carry):
        for r in range(BC):
            pltpu.sync_copy(ones, acc.at[dst_v.at[j, r]], add=True)
        return carry

    lax.fori_loop(0, NB, body, 0)
    plsc.subcore_barrier()
    pltpu.sync_copy(acc.at[pl.ds(sid * RPT, RPT)],
                    out_hbm.at[cid, pl.ds(sid * RPT, RPT)])


def _mm_body(deg_ref, x_ref, w_ref, y_ref):
    d = deg_ref[0] + deg_ref[1] + 1.0
    dinv = lax.rsqrt(jnp.maximum(d, 1e-12))
    xw = jnp.dot(x_ref[...], w_ref[...], preferred_element_type=jnp.float32)
    y_ref[...] = xw * dinv[:, None]


def _mm_tc(degp, xp, W):
    return pl.pallas_call(
        _mm_body,
        grid=(GRID,),
        in_specs=[
            pl.BlockSpec((NC, BLK), lambda i: (0, i)),
            pl.BlockSpec((BLK, D), lambda i: (i, 0)),
            pl.BlockSpec((D, D), lambda i: (0, 0)),
        ],
        out_specs=pl.BlockSpec((BLK, D), lambda i: (i, 0)),
        out_shape=jax.ShapeDtypeStruct((N, D), jnp.float32),
    )(degp, xp, W)


@functools.partial(
    pl.kernel,
    mesh=_mesh,
    out_type=jax.ShapeDtypeStruct((NC, NP, D), jnp.float32),
    scratch_types=[
        pltpu.VMEM((NB, BC, C), jnp.int32),
        pltpu.VMEM((BC, C), jnp.int32),
        pltpu.VMEM((BC, C), jnp.int32),
        pltpu.VMEM((C, D), jnp.float32),
        pltpu.VMEM((C, D), jnp.float32),
        pltpu.VMEM((16, D), jnp.float32),
        pltpu.VMEM_SHARED((NP, D), jnp.float32),
        pltpu.SemaphoreType.DMA,
        pltpu.SemaphoreType.DMA,
        pltpu.SemaphoreType.DMA,
        pltpu.SemaphoreType.DMA,
    ],
)
def _scat_sc(y_hbm, eS_hbm, out_hbm, src_v, dbuf_a, dbuf_b,
             rows_a, rows_b, zbuf, acc, sem_a, sem_b, dsem_a, dsem_b):
    cid = lax.axis_index("c")
    sid = lax.axis_index("s")
    wid = sid * NC + cid

    for i in range(16):
        for j in range(D // 16):
            zbuf[i, pl.ds(j * 16, 16)] = jnp.zeros((16,), jnp.float32)

    def zero_rows(i, carry):
        pltpu.sync_copy(zbuf, acc.at[pl.ds(sid * RPT + i * 16, 16)])
        return carry

    lax.fori_loop(0, RPT // 16, zero_rows, 0)
    pltpu.sync_copy(eS_hbm.at[0, wid], src_v)
    plsc.subcore_barrier()

    rows = (rows_a, rows_b)
    sems = (sem_a, sem_b)

    def gstart(a, r, par):
        pltpu.async_copy(y_hbm.at[src_v.at[a, r]], rows[par], sems[par])

    def gwait(a, r, par):
        pltpu.make_async_copy(y_hbm.at[src_v.at[a, r]], rows[par],
                              sems[par]).wait()

    def dstart(a, dbuf, dsem):
        pltpu.async_copy(eS_hbm.at[1, wid, a], dbuf, dsem)

    def dwait(a, dbuf, dsem):
        pltpu.make_async_copy(eS_hbm.at[1, wid, a], dbuf, dsem).wait()

    def run_block(a, dbuf):
        for k in range(BC):
            if k < BC - 1:
                gstart(a, k + 1, (k + 1) % 2)
            else:
                gstart(jnp.minimum(a + 1, NB - 1), 0, 0)
            gwait(a, k, k % 2)
            pltpu.sync_copy(rows[k % 2], acc.at[dbuf.at[k]], add=True)

    dstart(0, dbuf_a, dsem_a)
    dwait(0, dbuf_a, dsem_a)
    gstart(0, 0, 0)

    def body(j, carry):
        a0 = 2 * j
        dstart(a0 + 1, dbuf_b, dsem_b)
        run_block(a0, dbuf_a)
        dwait(a0 + 1, dbuf_b, dsem_b)
        dstart(jnp.minimum(a0 + 2, NB - 1), dbuf_a, dsem_a)
        run_block(a0 + 1, dbuf_b)
        dwait(jnp.minimum(a0 + 2, NB - 1), dbuf_a, dsem_a)
        return carry

    lax.fori_loop(0, NB // 2, body, 0)
    gwait(NB - 1, 0, 0)
    plsc.subcore_barrier()
    pltpu.sync_copy(acc.at[pl.ds(sid * RPT, RPT)],
                    out_hbm.at[cid, pl.ds(sid * RPT, RPT)])


def _fin_body(sp_ref, y_ref, deg_ref, b_ref, o_ref):
    d = deg_ref[0] + deg_ref[1] + 1.0
    dinv = lax.rsqrt(jnp.maximum(d, 1e-12))
    s = sp_ref[0] + sp_ref[1] + y_ref[...]
    o_ref[...] = dinv[:, None] * s + b_ref[0][None, :]


def _fin_tc(Sp, y, degp, b2):
    return pl.pallas_call(
        _fin_body,
        grid=(GRID,),
        in_specs=[
            pl.BlockSpec((NC, BLK, D), lambda i: (0, i, 0)),
            pl.BlockSpec((BLK, D), lambda i: (i, 0)),
            pl.BlockSpec((NC, BLK), lambda i: (0, i)),
            pl.BlockSpec((1, D), lambda i: (0, 0)),
        ],
        out_specs=pl.BlockSpec((BLK, D), lambda i: (i, 0)),
        out_shape=jax.ShapeDtypeStruct((N, D), jnp.float32),
    )(Sp, y, degp, b2)


def kernel(x, edge_index, W, b):
    eS = edge_index.reshape(2, NTILES, NB, BC, C)
    degp = _deg_sc(eS)
    y = _mm_tc(degp, x, W)
    Sp = _scat_sc(y, eS)
    return _fin_tc(Sp, y, degp, b.reshape(1, D))
